# Initial kernel scaffold; baseline (speedup 1.0000x reference)
import jax
import jax.numpy as jnp
from jax.experimental import pallas as pl


def kernel(sequence, segment_label, seg_table, pos_pe):
    raise NotImplementedError("write your pallas kernel here")


# speedup vs baseline: 8.4423x; 8.4423x over previous
import jax
import jax.numpy as jnp
from jax.experimental import pallas as pl


def _body(seq_ref, lab_ref, tab_ref, pe_ref, out_ref):
    lab = lab_ref[...]
    tab = tab_ref[...]
    t0 = tab[0][None, None, :]
    t1 = tab[1][None, None, :]
    t2 = tab[2][None, None, :]
    lab3 = lab[:, :, None]
    seg = jnp.where(lab3 == 1, t1, jnp.where(lab3 == 2, t2, t0))
    out_ref[...] = seq_ref[...] + pe_ref[...][None, :, :] + seg


def kernel(sequence, segment_label, seg_table, pos_pe):
    B, L, D = sequence.shape
    pe = pos_pe.reshape(L, D)
    bB = 16
    return pl.pallas_call(
        _body,
        grid=(B // bB,),
        in_specs=[
            pl.BlockSpec((bB, L, D), lambda i: (i, 0, 0)),
            pl.BlockSpec((bB, L), lambda i: (i, 0)),
            pl.BlockSpec((3, D), lambda i: (0, 0)),
            pl.BlockSpec((L, D), lambda i: (0, 0)),
        ],
        out_specs=pl.BlockSpec((bB, L, D), lambda i: (i, 0, 0)),
        out_shape=jax.ShapeDtypeStruct((B, L, D), jnp.float32),
    )(sequence, segment_label, seg_table, pe)
